# baseline (speedup 1.0000x reference)
import math

import jax
import jax.numpy as jnp
from jax import lax
from jax.experimental import pallas as pl
from jax.experimental.pallas import tpu as pltpu
from jax.experimental.pallas import tpu_sc as plsc

N_ATOMS = 10000
N_EDGES = 320000
D = 128
D_RBF = 16
INV_SQRT_2 = 1.0 / math.sqrt(2.0)


_BG = 12800


def _stage1_body(m_ref, bt_ref, w_ref, x_ref):
    emb = lax.dot_general(bt_ref[...], w_ref[...],
                          dimension_numbers=(((0,), (0,)), ((), ())),
                          preferred_element_type=jnp.float32)
    x_ref[...] = m_ref[...] * emb


def _stage1(m, basis_t, w_eff, n_rows, blk_off):
    grid = n_rows // _BG
    return pl.pallas_call(
        _stage1_body,
        grid=(grid,),
        in_specs=[
            pl.BlockSpec((_BG, D), lambda g: (g + blk_off, 0)),
            pl.BlockSpec((D_RBF, _BG), lambda g: (0, g + blk_off)),
            pl.BlockSpec((D_RBF, D), lambda g: (0, 0)),
        ],
        out_specs=pl.BlockSpec((_BG, D), lambda g: (g, 0)),
        out_shape=jax.ShapeDtypeStruct((n_rows, D), jnp.float32),
    )(m, basis_t, w_eff)


_E_TC = 204800
_BT = 1024
_N_PAD = 10240


def _stage1a_body(wstart_ref, wcnt_ref, m_ref, bt_ref, w_ref, idx_ref, o_ref,
                  acc_ref):
    g = pl.program_id(0)
    ng = pl.num_programs(0)

    @pl.when(g == 0)
    def _():
        acc_ref[...] = jnp.zeros((_N_PAD, D), jnp.float32)

    emb = lax.dot_general(bt_ref[...], w_ref[...],
                          dimension_numbers=(((0,), (0,)), ((), ())),
                          preferred_element_type=jnp.float32)
    xb = (m_ref[...] * emb).astype(jnp.bfloat16)
    idxv = idx_ref[0, 0, :]
    iota_sub = lax.broadcasted_iota(jnp.int32, (128, _BT), 0)
    w0 = wstart_ref[g]
    nw = wcnt_ref[g]

    def wbody(k, carry):
        wo = pl.multiple_of((w0 + k) * 128, 8)
        onehot = (iota_sub + (w0 + k) * 128 == idxv[None, :]).astype(jnp.bfloat16)
        part = lax.dot_general(onehot, xb,
                               dimension_numbers=(((1,), (0,)), ((), ())),
                               preferred_element_type=jnp.float32)
        acc_ref[pl.ds(wo, 128), :] += part
        return carry

    lax.fori_loop(0, nw, wbody, 0)

    @pl.when(g == ng - 1)
    def _():
        o_ref[...] = acc_ref[...]


def _stage1a(m, basis_t, w_eff, idx3):
    grid = _E_TC // _BT
    gspec = pltpu.PrefetchScalarGridSpec(
        num_scalar_prefetch=2,
        grid=(grid,),
        in_specs=[
            pl.BlockSpec((_BT, D), lambda g, s0, s1: (g, 0)),
            pl.BlockSpec((D_RBF, _BT), lambda g, s0, s1: (0, g)),
            pl.BlockSpec((D_RBF, D), lambda g, s0, s1: (0, 0)),
            pl.BlockSpec((1, 1, _BT), lambda g, s0, s1: (g, 0, 0)),
        ],
        out_specs=pl.BlockSpec((_N_PAD, D), lambda g, s0, s1: (0, 0)),
        scratch_shapes=[pltpu.VMEM((_N_PAD, D), jnp.float32)],
    )
    idx_blk = idx3[:_E_TC].reshape(grid, _BT)
    first = idx_blk[:, 0] // 128
    last = idx_blk[:, _BT - 1] // 128
    wcnt = last - first + 1
    return pl.pallas_call(
        _stage1a_body,
        grid_spec=gspec,
        out_shape=jax.ShapeDtypeStruct((_N_PAD, D), jnp.float32),
    )(first, wcnt, m, basis_t, w_eff, idx_blk.reshape(grid, 1, _BT))


_C = 128
_NCHUNKS = N_EDGES // _C
_E_SC = N_EDGES - _E_TC
_SC_OFF = _E_TC // _C
_NBLK = _E_SC // _C
_NW = 32
_IDXROWS = 88
_IDXPAD_ROWS = 2552
_ROWS_PER_SUB = _N_PAD // 16


def _make_sc_body(chunk_off):
    def _sc_scatter_body(x_hbm, idx2_hbm, zeros_hbm, out_hbm, xb0, xb1, idxv,
                         acc, sem0, sem1):
        c = lax.axis_index("c")
        s = lax.axis_index("s")
        wid = c * 16 + s

        pltpu.sync_copy(zeros_hbm.at[pl.ds(0, _ROWS_PER_SUB)],
                        acc.at[pl.ds(s * _ROWS_PER_SUB, _ROWS_PER_SUB)])

        lo = (wid * _NBLK) // _NW
        hi = ((wid + 1) * _NBLK) // _NW
        t = hi - lo
        cbase = chunk_off + lo
        a0 = pl.multiple_of((cbase >> 3) << 3, 8)
        ishift = cbase - a0
        pltpu.sync_copy(idx2_hbm.at[pl.ds(a0, _IDXROWS)], idxv)

        def start(buf, sem, i):
            off = pl.multiple_of((lo + i) * _C, 8)
            pltpu.async_copy(x_hbm.at[pl.ds(off, _C)], buf, sem)

        def wait(buf, sem, i):
            off = pl.multiple_of((lo + i) * _C, 8)
            pltpu.make_async_copy(x_hbm.at[pl.ds(off, _C)], buf, sem).wait()

        def scat(buf, i):
            pltpu.sync_copy(buf, acc.at[idxv.at[ishift + i]], add=True)

        plsc.subcore_barrier()
        start(xb0, sem0, 0)

        def pair(p, carry):
            i0 = 2 * p
            i1 = i0 + 1

            @pl.when(i1 < t)
            def _():
                start(xb1, sem1, i1)

            wait(xb0, sem0, i0)
            scat(xb0, i0)

            @pl.when(i1 < t)
            def _():
                @pl.when(i1 + 1 < t)
                def _():
                    start(xb0, sem0, i1 + 1)

                wait(xb1, sem1, i1)
                scat(xb1, i1)

            return carry

        lax.fori_loop(0, (t + 1) // 2, pair, 0)
        plsc.subcore_barrier()

        pltpu.sync_copy(acc.at[pl.ds(s * _ROWS_PER_SUB, _ROWS_PER_SUB)],
                        out_hbm.at[c].at[pl.ds(s * _ROWS_PER_SUB, _ROWS_PER_SUB)])

    return _sc_scatter_body


def _stage2(x_part, idx2, zeros_rows, chunk_off):
    mesh = plsc.VectorSubcoreMesh(core_axis_name="c", subcore_axis_name="s")
    f = pl.kernel(
        _make_sc_body(chunk_off),
        out_type=jax.ShapeDtypeStruct((2, _N_PAD, D), jnp.float32),
        mesh=mesh,
        scratch_types=[
            pltpu.VMEM((_C, D), jnp.float32),
            pltpu.VMEM((_C, D), jnp.float32),
            pltpu.VMEM((_IDXROWS, _C), jnp.int32),
            pltpu.VMEM_SHARED((_N_PAD, D), jnp.float32),
            pltpu.SemaphoreType.DMA,
            pltpu.SemaphoreType.DMA,
        ],
    )
    return f(x_part, idx2, zeros_rows)


_BA = 2000


def _ssilu(x):
    sig = 1.0 / (1.0 + jnp.exp(-x))
    return x * sig * (1.0 / 0.6)


def _stage3_body(psc_ref, ptc_ref, wa0_ref, wb0_ref, wa1_ref, wb1_ref, o_ref):
    x = psc_ref[0] + psc_ref[1] + ptc_ref[...]
    for wa, wb in ((wa0_ref, wb0_ref), (wa1_ref, wb1_ref)):
        y = _ssilu(jnp.dot(x, wa[...], preferred_element_type=jnp.float32))
        y = _ssilu(jnp.dot(y, wb[...], preferred_element_type=jnp.float32))
        x = (x + y) * INV_SQRT_2
    o_ref[...] = x


def _stage3(p_sc, p_tc, wa0, wb0, wa1, wb1):
    grid = N_ATOMS // _BA
    wspec = pl.BlockSpec((D, D), lambda g: (0, 0))
    return pl.pallas_call(
        _stage3_body,
        grid=(grid,),
        in_specs=[
            pl.BlockSpec((2, _BA, D), lambda g: (0, g, 0)),
            pl.BlockSpec((_BA, D), lambda g: (g, 0)),
            wspec, wspec, wspec, wspec,
        ],
        out_specs=pl.BlockSpec((_BA, D), lambda g: (g, 0)),
        out_shape=jax.ShapeDtypeStruct((N_ATOMS, D), jnp.float32),
    )(p_sc, p_tc, wa0, wb0, wa1, wb1)


def kernel(h, m, basis_rad, idx_atom, W_rbf, scale_sum, W_r0a, W_r0b, W_r1a, W_r1b):
    del h
    w_eff = W_rbf * scale_sum
    basis_t = basis_rad.T
    idx2 = jnp.concatenate(
        [idx_atom, jnp.zeros((_IDXPAD_ROWS * _C - N_EDGES,), jnp.int32)]
    ).reshape(_IDXPAD_ROWS, _C)
    zeros_rows = jnp.zeros((_ROWS_PER_SUB, D), jnp.float32)
    x_sc = _stage1(m, basis_t, w_eff, _E_SC, _E_TC // _BG)
    p_sc = _stage2(x_sc, idx2, zeros_rows, _SC_OFF)
    p_tc = _stage1a(m, basis_t, w_eff, idx_atom)
    return _stage3(p_sc, p_tc, W_r0a, W_r0b, W_r1a, W_r1b)


# speedup vs baseline: 1.2396x; 1.2396x over previous
import math

import jax
import jax.numpy as jnp
from jax import lax
from jax.experimental import pallas as pl
from jax.experimental.pallas import tpu as pltpu
from jax.experimental.pallas import tpu_sc as plsc

N_ATOMS = 10000
N_EDGES = 320000
D = 128
D_RBF = 16
INV_SQRT_2 = 1.0 / math.sqrt(2.0)


_BG = 12800


def _stage1_body(m_ref, bt_ref, w_ref, x_ref):
    emb = lax.dot_general(bt_ref[...], w_ref[...],
                          dimension_numbers=(((0,), (0,)), ((), ())),
                          preferred_element_type=jnp.float32)
    x_ref[...] = m_ref[...] * emb


def _stage1(m, basis_t, w_eff, n_rows, blk_off):
    grid = n_rows // _BG
    return pl.pallas_call(
        _stage1_body,
        grid=(grid,),
        in_specs=[
            pl.BlockSpec((_BG, D), lambda g: (g + blk_off, 0)),
            pl.BlockSpec((D_RBF, _BG), lambda g: (0, g + blk_off)),
            pl.BlockSpec((D_RBF, D), lambda g: (0, 0)),
        ],
        out_specs=pl.BlockSpec((_BG, D), lambda g: (g, 0)),
        out_shape=jax.ShapeDtypeStruct((n_rows, D), jnp.float32),
    )(m, basis_t, w_eff)


_E_TC = 102400
_BT = 1024
_N_PAD = 10240


def _stage1a_body(wstart_ref, wcnt_ref, m_ref, bt_ref, w_ref, idx_ref, o_ref,
                  acc_ref):
    g = pl.program_id(0)
    ng = pl.num_programs(0)

    @pl.when(g == 0)
    def _():
        acc_ref[...] = jnp.zeros((_N_PAD, D), jnp.float32)

    emb = lax.dot_general(bt_ref[...], w_ref[...],
                          dimension_numbers=(((0,), (0,)), ((), ())),
                          preferred_element_type=jnp.float32)
    xb = (m_ref[...] * emb).astype(jnp.bfloat16)
    idxv = idx_ref[0, 0, :]
    iota_sub = lax.broadcasted_iota(jnp.int32, (128, _BT), 0)
    w0 = wstart_ref[g]
    nw = wcnt_ref[g]

    def wbody(k, carry):
        wo = pl.multiple_of((w0 + k) * 128, 8)
        onehot = (iota_sub + (w0 + k) * 128 == idxv[None, :]).astype(jnp.bfloat16)
        part = lax.dot_general(onehot, xb,
                               dimension_numbers=(((1,), (0,)), ((), ())),
                               preferred_element_type=jnp.float32)
        acc_ref[pl.ds(wo, 128), :] += part
        return carry

    lax.fori_loop(0, nw, wbody, 0)

    @pl.when(g == ng - 1)
    def _():
        o_ref[...] = acc_ref[...]


def _stage1a(m, basis_t, w_eff, idx3):
    grid = _E_TC // _BT
    gspec = pltpu.PrefetchScalarGridSpec(
        num_scalar_prefetch=2,
        grid=(grid,),
        in_specs=[
            pl.BlockSpec((_BT, D), lambda g, s0, s1: (g, 0)),
            pl.BlockSpec((D_RBF, _BT), lambda g, s0, s1: (0, g)),
            pl.BlockSpec((D_RBF, D), lambda g, s0, s1: (0, 0)),
            pl.BlockSpec((1, 1, _BT), lambda g, s0, s1: (g, 0, 0)),
        ],
        out_specs=pl.BlockSpec((_N_PAD, D), lambda g, s0, s1: (0, 0)),
        scratch_shapes=[pltpu.VMEM((_N_PAD, D), jnp.float32)],
    )
    idx_blk = idx3[:_E_TC].reshape(grid, _BT)
    first = idx_blk[:, 0] // 128
    last = idx_blk[:, _BT - 1] // 128
    wcnt = last - first + 1
    return pl.pallas_call(
        _stage1a_body,
        grid_spec=gspec,
        out_shape=jax.ShapeDtypeStruct((_N_PAD, D), jnp.float32),
    )(first, wcnt, m, basis_t, w_eff, idx_blk.reshape(grid, 1, _BT))


_C = 128
_NCHUNKS = N_EDGES // _C
_E_SC = N_EDGES - _E_TC
_SC_OFF = _E_TC // _C
_NBLK = _E_SC // _C
_NW = 32
_IDXROWS = 88
_IDXPAD_ROWS = 2552
_ROWS_PER_SUB = _N_PAD // 16


def _make_sc_body(chunk_off):
    def _sc_scatter_body(x_hbm, idx2_hbm, zeros_hbm, out_hbm, xb0, xb1, idxv,
                         acc, sem0, sem1):
        c = lax.axis_index("c")
        s = lax.axis_index("s")
        wid = c * 16 + s

        pltpu.sync_copy(zeros_hbm.at[pl.ds(0, _ROWS_PER_SUB)],
                        acc.at[pl.ds(s * _ROWS_PER_SUB, _ROWS_PER_SUB)])

        lo = (wid * _NBLK) // _NW
        hi = ((wid + 1) * _NBLK) // _NW
        t = hi - lo
        cbase = chunk_off + lo
        a0 = pl.multiple_of((cbase >> 3) << 3, 8)
        ishift = cbase - a0
        pltpu.sync_copy(idx2_hbm.at[pl.ds(a0, _IDXROWS)], idxv)

        def start(buf, sem, i):
            off = pl.multiple_of((lo + i) * _C, 8)
            pltpu.async_copy(x_hbm.at[pl.ds(off, _C)], buf, sem)

        def wait(buf, sem, i):
            off = pl.multiple_of((lo + i) * _C, 8)
            pltpu.make_async_copy(x_hbm.at[pl.ds(off, _C)], buf, sem).wait()

        def scat(buf, i):
            pltpu.sync_copy(buf, acc.at[idxv.at[ishift + i]], add=True)

        plsc.subcore_barrier()
        start(xb0, sem0, 0)

        def pair(p, carry):
            i0 = 2 * p
            i1 = i0 + 1

            @pl.when(i1 < t)
            def _():
                start(xb1, sem1, i1)

            wait(xb0, sem0, i0)
            scat(xb0, i0)

            @pl.when(i1 < t)
            def _():
                @pl.when(i1 + 1 < t)
                def _():
                    start(xb0, sem0, i1 + 1)

                wait(xb1, sem1, i1)
                scat(xb1, i1)

            return carry

        lax.fori_loop(0, (t + 1) // 2, pair, 0)
        plsc.subcore_barrier()

        pltpu.sync_copy(acc.at[pl.ds(s * _ROWS_PER_SUB, _ROWS_PER_SUB)],
                        out_hbm.at[c].at[pl.ds(s * _ROWS_PER_SUB, _ROWS_PER_SUB)])

    return _sc_scatter_body


def _stage2(x_part, idx2, zeros_rows, chunk_off):
    mesh = plsc.VectorSubcoreMesh(core_axis_name="c", subcore_axis_name="s")
    f = pl.kernel(
        _make_sc_body(chunk_off),
        out_type=jax.ShapeDtypeStruct((2, _N_PAD, D), jnp.float32),
        mesh=mesh,
        scratch_types=[
            pltpu.VMEM((_C, D), jnp.float32),
            pltpu.VMEM((_C, D), jnp.float32),
            pltpu.VMEM((_IDXROWS, _C), jnp.int32),
            pltpu.VMEM_SHARED((_N_PAD, D), jnp.float32),
            pltpu.SemaphoreType.DMA,
            pltpu.SemaphoreType.DMA,
        ],
    )
    return f(x_part, idx2, zeros_rows)


_BA = 2000


def _ssilu(x):
    sig = 1.0 / (1.0 + jnp.exp(-x))
    return x * sig * (1.0 / 0.6)


def _stage3_body(psc_ref, ptc_ref, wa0_ref, wb0_ref, wa1_ref, wb1_ref, o_ref):
    x = psc_ref[0] + psc_ref[1] + ptc_ref[...]
    for wa, wb in ((wa0_ref, wb0_ref), (wa1_ref, wb1_ref)):
        y = _ssilu(jnp.dot(x, wa[...], preferred_element_type=jnp.float32))
        y = _ssilu(jnp.dot(y, wb[...], preferred_element_type=jnp.float32))
        x = (x + y) * INV_SQRT_2
    o_ref[...] = x


def _stage3(p_sc, p_tc, wa0, wb0, wa1, wb1):
    grid = N_ATOMS // _BA
    wspec = pl.BlockSpec((D, D), lambda g: (0, 0))
    return pl.pallas_call(
        _stage3_body,
        grid=(grid,),
        in_specs=[
            pl.BlockSpec((2, _BA, D), lambda g: (0, g, 0)),
            pl.BlockSpec((_BA, D), lambda g: (g, 0)),
            wspec, wspec, wspec, wspec,
        ],
        out_specs=pl.BlockSpec((_BA, D), lambda g: (g, 0)),
        out_shape=jax.ShapeDtypeStruct((N_ATOMS, D), jnp.float32),
    )(p_sc, p_tc, wa0, wb0, wa1, wb1)


def kernel(h, m, basis_rad, idx_atom, W_rbf, scale_sum, W_r0a, W_r0b, W_r1a, W_r1b):
    del h
    w_eff = W_rbf * scale_sum
    basis_t = basis_rad.T
    idx2 = jnp.concatenate(
        [idx_atom, jnp.zeros((_IDXPAD_ROWS * _C - N_EDGES,), jnp.int32)]
    ).reshape(_IDXPAD_ROWS, _C)
    zeros_rows = jnp.zeros((_ROWS_PER_SUB, D), jnp.float32)
    x_sc = _stage1(m, basis_t, w_eff, _E_SC, _E_TC // _BG)
    p_sc = _stage2(x_sc, idx2, zeros_rows, _SC_OFF)
    p_tc = _stage1a(m, basis_t, w_eff, idx_atom)
    return _stage3(p_sc, p_tc, W_r0a, W_r0b, W_r1a, W_r1b)
